# Initial kernel scaffold; baseline (speedup 1.0000x reference)
import jax
import jax.numpy as jnp
from jax.experimental import pallas as pl


def kernel(x, edge_index1, edge_index2, batch, W1, b1, W2, b2, Wfc, bfc):
    raise NotImplementedError("write your pallas kernel here")


# speedup vs baseline: 18.6522x; 18.6522x over previous
import functools

import jax
import jax.numpy as jnp
from jax import lax
from jax.experimental import pallas as pl
from jax.experimental.pallas import tpu as pltpu
from jax.experimental.pallas import tpu_sc as plsc

N = 10000
D = 128
H = 64
G = 64
NCLS = 32
E = 320000
LANES = 128
NSUB = 16
CPT = 160
NSLAB = 5
SCH = CPT // NSLAB
CPS = CPT * NSUB
PADE = CPS * LANES - E
ACC_ROWS = NSUB * 640
RPT = 624
DW = 128

@functools.cache
def _get_deg_kernel():
    mesh = plsc.VectorSubcoreMesh(core_axis_name="c", subcore_axis_name="s")

    @functools.partial(
        pl.kernel,
        out_type=jax.ShapeDtypeStruct((2 * N, DW), jnp.float32),
        mesh=mesh,
        scratch_types=[
            pltpu.VMEM((CPT, LANES), jnp.int32),
            pltpu.VMEM((LANES, DW), jnp.float32),
            pltpu.VMEM_SHARED((ACC_ROWS, DW), jnp.float32),
        ],
    )
    def _deg_kernel(col_hbm, ones_hbm, zeros_hbm, out_hbm, col_v, ones_v, acc_sh):
        c = lax.axis_index("c")
        s = lax.axis_index("s")
        pltpu.sync_copy(col_hbm.at[pl.ds(c * CPS + s * CPT, CPT)], col_v)
        pltpu.sync_copy(ones_hbm, ones_v)
        pltpu.sync_copy(zeros_hbm, acc_sh.at[pl.ds(s * (ACC_ROWS // NSUB), ACC_ROWS // NSUB)])
        plsc.subcore_barrier()

        @pl.loop(0, CPT)
        def _(j):
            pltpu.sync_copy(ones_v, acc_sh.at[col_v.at[j]], add=True)

        plsc.subcore_barrier()
        pltpu.sync_copy(acc_sh.at[pl.ds(s * RPT, RPT)],
                        out_hbm.at[pl.ds(c * N + s * RPT, RPT)])

        @pl.when(s == 0)
        def _():
            pltpu.sync_copy(acc_sh.at[pl.ds(NSUB * RPT, N - NSUB * RPT)],
                            out_hbm.at[pl.ds(c * N + NSUB * RPT, N - NSUB * RPT)])

    return _deg_kernel


@functools.cache
def _make_agg(W):
    mesh = plsc.VectorSubcoreMesh(core_axis_name="c", subcore_axis_name="s")

    @functools.partial(
        pl.kernel,
        out_type=jax.ShapeDtypeStruct((2 * N, W), jnp.float32),
        mesh=mesh,
        scratch_types=[
            pltpu.VMEM((SCH, LANES), jnp.int32),
            pltpu.VMEM((SCH, LANES), jnp.int32),
            pltpu.VMEM((LANES, W), jnp.float32),
            pltpu.VMEM_SHARED((ACC_ROWS, W), jnp.float32),
            pltpu.SemaphoreType.DMA,
        ],
    )
    def _agg(p_hbm, row_hbm, col_hbm, zeros_hbm, out_hbm,
             row_v, col_v, rows_v, acc_sh, sem):
        c = lax.axis_index("c")
        s = lax.axis_index("s")
        base = c * CPS + s * CPT
        pltpu.sync_copy(zeros_hbm, acc_sh.at[pl.ds(s * (ACC_ROWS // NSUB), ACC_ROWS // NSUB)])
        plsc.subcore_barrier()

        @pl.loop(0, NSLAB)
        def _(t):
            pltpu.sync_copy(row_hbm.at[pl.ds(base + t * SCH, SCH)], row_v)
            pltpu.sync_copy(col_hbm.at[pl.ds(base + t * SCH, SCH)], col_v)

            @pl.loop(0, SCH)
            def _(j):
                pltpu.async_copy(p_hbm.at[row_v.at[j]], rows_v, sem).wait()
                pltpu.sync_copy(rows_v, acc_sh.at[col_v.at[j]], add=True)

        plsc.subcore_barrier()
        pltpu.sync_copy(acc_sh.at[pl.ds(s * RPT, RPT)],
                        out_hbm.at[pl.ds(c * N + s * RPT, RPT)])

        @pl.when(s == 0)
        def _():
            pltpu.sync_copy(acc_sh.at[pl.ds(NSUB * RPT, N - NSUB * RPT)],
                            out_hbm.at[pl.ds(c * N + NSUB * RPT, N - NSUB * RPT)])

    return _agg


def _prep_body(x_ref, w1_ref, deg_ref, p_ref):
    h = jnp.dot(x_ref[...], w1_ref[...], preferred_element_type=jnp.float32)
    deg = deg_ref[...]
    dinv1 = lax.rsqrt(deg[0:N, 0:1] + 1.0)
    dinv2 = lax.rsqrt(deg[N:2 * N, 0:1] + 1.0)
    p_ref[0:N, 0:H] = h * dinv1
    p_ref[0:N, H:2 * H] = h * dinv2


def _mid_body(acc_ref, p_ref, deg_ref, b1_ref, w2_ref, q_ref):
    deg = deg_ref[...]
    dinv1 = lax.rsqrt(deg[0:N, 0:1] + 1.0)
    dinv2 = lax.rsqrt(deg[N:2 * N, 0:1] + 1.0)
    p = p_ref[...]
    acc = acc_ref[...]
    b1 = b1_ref[...]
    x1 = jnp.maximum(dinv1 * (acc[0:N, 0:H] + p[0:N, 0:H]) + b1, 0.0)
    x2 = jnp.maximum(dinv2 * (acc[N:2 * N, H:2 * H] + p[0:N, H:2 * H]) + b1, 0.0)
    hcat = jnp.concatenate([x1, x2], axis=1)
    g = jnp.dot(hcat, w2_ref[...], preferred_element_type=jnp.float32)
    q_ref[0:N, :] = g * dinv1
    q_ref[N:2 * N, :] = g * dinv2


def _fin_body(acc_ref, q_ref, deg_ref, b2_ref, h2_ref):
    deg = deg_ref[...]
    dinv1 = lax.rsqrt(deg[0:N, 0:1] + 1.0)
    dinv2 = lax.rsqrt(deg[N:2 * N, 0:1] + 1.0)
    q = q_ref[...]
    acc = acc_ref[...]
    b2 = b2_ref[...]
    y1 = jnp.maximum(dinv1 * (acc[0:N] + q[0:N]) + b2, 0.0)
    y2 = jnp.maximum(dinv2 * (acc[N:2 * N] + q[N:2 * N]) + b2, 0.0)
    h2_ref[0:N, 0:2 * H] = y1
    h2_ref[0:N, 2 * H:4 * H] = y2


def _pool_body(h2_ref, batch_ref, wfc_ref, bfc_ref, o_ref):
    h2 = h2_ref[...]
    b = batch_ref[...]
    oh = (b == lax.broadcasted_iota(jnp.int32, (N, G), 1)).astype(jnp.float32)
    sums = lax.dot_general(oh, h2, (((0,), (0,)), ((), ())),
                           preferred_element_type=jnp.float32)
    counts = jnp.sum(oh, axis=0)
    pooled = sums / jnp.maximum(counts, 1.0)[:, None]
    o_ref[...] = jnp.dot(pooled, wfc_ref[...],
                         preferred_element_type=jnp.float32) + bfc_ref[...]


def kernel(x, edge_index1, edge_index2, batch, W1, b1, W2, b2, Wfc, bfc):
    ei1 = edge_index1.astype(jnp.int32)
    ei2 = edge_index2.astype(jnp.int32)
    pad_i = jnp.arange(PADE, dtype=jnp.int32)
    pad_row = pad_i % N
    pad_col = N + pad_i % (ACC_ROWS - N)
    row1_flat = jnp.concatenate([ei1[0], pad_row, ei2[0], pad_row])
    row2_flat = jnp.concatenate([ei1[0], pad_row, ei2[0] + N, pad_row + N])
    col_flat = jnp.concatenate([ei1[1], pad_col, ei2[1], pad_col])
    row1_hbm = row1_flat.reshape(2 * CPS, LANES)
    row2_hbm = row2_flat.reshape(2 * CPS, LANES)
    col_hbm = col_flat.reshape(2 * CPS, LANES)
    ones_h = jnp.ones((LANES, DW), jnp.float32)
    zer_deg = jnp.zeros((ACC_ROWS // NSUB, DW), jnp.float32)
    zer128 = jnp.zeros((ACC_ROWS // NSUB, 2 * H), jnp.float32)

    deg = _get_deg_kernel()(col_hbm, ones_h, zer_deg)
    p = pl.pallas_call(
        _prep_body,
        out_shape=jax.ShapeDtypeStruct((N, 2 * H), jnp.float32),
    )(x, W1, deg)
    acc1 = _make_agg(2 * H)(p, row1_hbm, col_hbm, zer128)
    q = pl.pallas_call(
        _mid_body,
        out_shape=jax.ShapeDtypeStruct((2 * N, 2 * H), jnp.float32),
    )(acc1, p, deg, b1.reshape(1, H), W2)
    acc2 = _make_agg(2 * H)(q, row2_hbm, col_hbm, zer128)
    h2 = pl.pallas_call(
        _fin_body,
        out_shape=jax.ShapeDtypeStruct((N, 4 * H), jnp.float32),
    )(acc2, q, deg, b2.reshape(1, 2 * H))
    out = pl.pallas_call(
        _pool_body,
        out_shape=jax.ShapeDtypeStruct((G, NCLS), jnp.float32),
    )(h2, batch.reshape(N, 1).astype(jnp.int32), Wfc, bfc.reshape(1, NCLS))
    return out


# speedup vs baseline: 23.4439x; 1.2569x over previous
import functools

import jax
import jax.numpy as jnp
from jax import lax
from jax.experimental import pallas as pl
from jax.experimental.pallas import tpu as pltpu
from jax.experimental.pallas import tpu_sc as plsc

N = 10000
D = 128
H = 64
G = 64
NCLS = 32
E = 320000
LANES = 128
NSUB = 16
CPT = 160
NSLAB = 5
SCH = CPT // NSLAB
CPS = CPT * NSUB
PADE = CPS * LANES - E
ACC_ROWS = NSUB * 640
RPT = 624
DW = 128

@functools.cache
def _get_deg_kernel():
    mesh = plsc.VectorSubcoreMesh(core_axis_name="c", subcore_axis_name="s")

    @functools.partial(
        pl.kernel,
        out_type=jax.ShapeDtypeStruct((2 * N, DW), jnp.float32),
        mesh=mesh,
        scratch_types=[
            pltpu.VMEM((CPT, LANES), jnp.int32),
            pltpu.VMEM((LANES, DW), jnp.float32),
            pltpu.VMEM_SHARED((ACC_ROWS, DW), jnp.float32),
            pltpu.SemaphoreType.DMA,
        ],
    )
    def _deg_kernel(col_hbm, ones_hbm, zeros_hbm, out_hbm, col_v, ones_v,
                    acc_sh, sem):
        c = lax.axis_index("c")
        s = lax.axis_index("s")
        pltpu.sync_copy(col_hbm.at[pl.ds(c * CPS + s * CPT, CPT)], col_v)
        pltpu.sync_copy(ones_hbm, ones_v)
        pltpu.sync_copy(zeros_hbm, acc_sh.at[pl.ds(s * (ACC_ROWS // NSUB), ACC_ROWS // NSUB)])
        plsc.subcore_barrier()

        KF = 16

        @pl.loop(0, CPT // KF)
        def _(b):
            @pl.loop(0, KF)
            def _(i):
                pltpu.async_copy(ones_v, acc_sh.at[col_v.at[b * KF + i]],
                                 sem, add=True)

            @pl.loop(0, KF)
            def _(i):
                pltpu.make_async_copy(ones_v, acc_sh.at[col_v.at[b * KF + i]],
                                      sem).wait()

        plsc.subcore_barrier()
        pltpu.sync_copy(acc_sh.at[pl.ds(s * RPT, RPT)],
                        out_hbm.at[pl.ds(c * N + s * RPT, RPT)])

        @pl.when(s == 0)
        def _():
            pltpu.sync_copy(acc_sh.at[pl.ds(NSUB * RPT, N - NSUB * RPT)],
                            out_hbm.at[pl.ds(c * N + NSUB * RPT, N - NSUB * RPT)])

    return _deg_kernel


@functools.cache
def _make_agg(W):
    mesh = plsc.VectorSubcoreMesh(core_axis_name="c", subcore_axis_name="s")

    @functools.partial(
        pl.kernel,
        out_type=jax.ShapeDtypeStruct((2 * N, W), jnp.float32),
        mesh=mesh,
        scratch_types=[
            pltpu.VMEM((SCH, LANES), jnp.int32),
            pltpu.VMEM((SCH, LANES), jnp.int32),
            pltpu.VMEM((LANES, W), jnp.float32),
            pltpu.VMEM((LANES, W), jnp.float32),
            pltpu.VMEM_SHARED((ACC_ROWS, W), jnp.float32),
            pltpu.SemaphoreType.DMA,
        ],
    )
    def _agg(p_hbm, row_hbm, col_hbm, zeros_hbm, out_hbm,
             row_v, col_v, rows_a, rows_b, acc_sh, sem):
        c = lax.axis_index("c")
        s = lax.axis_index("s")
        base = c * CPS + s * CPT
        pltpu.sync_copy(zeros_hbm, acc_sh.at[pl.ds(s * (ACC_ROWS // NSUB), ACC_ROWS // NSUB)])
        plsc.subcore_barrier()

        @pl.loop(0, NSLAB)
        def _(t):
            pltpu.sync_copy(row_hbm.at[pl.ds(base + t * SCH, SCH)], row_v)
            pltpu.sync_copy(col_hbm.at[pl.ds(base + t * SCH, SCH)], col_v)
            pltpu.async_copy(p_hbm.at[row_v.at[0]], rows_a, sem).wait()

            @pl.loop(0, SCH // 2)
            def _(u):
                j0 = 2 * u
                hb = pltpu.async_copy(p_hbm.at[row_v.at[j0 + 1]], rows_b, sem)
                pltpu.sync_copy(rows_a, acc_sh.at[col_v.at[j0]], add=True)
                hb.wait()

                @pl.when(u < SCH // 2 - 1)
                def _():
                    pltpu.async_copy(p_hbm.at[row_v.at[j0 + 2]], rows_a, sem)

                pltpu.sync_copy(rows_b, acc_sh.at[col_v.at[j0 + 1]], add=True)

                @pl.when(u < SCH // 2 - 1)
                def _():
                    pltpu.make_async_copy(
                        p_hbm.at[row_v.at[j0 + 2]], rows_a, sem).wait()

        plsc.subcore_barrier()
        pltpu.sync_copy(acc_sh.at[pl.ds(s * RPT, RPT)],
                        out_hbm.at[pl.ds(c * N + s * RPT, RPT)])

        @pl.when(s == 0)
        def _():
            pltpu.sync_copy(acc_sh.at[pl.ds(NSUB * RPT, N - NSUB * RPT)],
                            out_hbm.at[pl.ds(c * N + NSUB * RPT, N - NSUB * RPT)])

    return _agg


def _prep_body(x_ref, w1_ref, deg_ref, p_ref):
    h = jnp.dot(x_ref[...], w1_ref[...], preferred_element_type=jnp.float32)
    deg = deg_ref[...]
    dinv1 = lax.rsqrt(deg[0:N, 0:1] + 1.0)
    dinv2 = lax.rsqrt(deg[N:2 * N, 0:1] + 1.0)
    p_ref[0:N, 0:H] = h * dinv1
    p_ref[0:N, H:2 * H] = h * dinv2


def _mid_body(acc_ref, p_ref, deg_ref, b1_ref, w2_ref, q_ref):
    deg = deg_ref[...]
    dinv1 = lax.rsqrt(deg[0:N, 0:1] + 1.0)
    dinv2 = lax.rsqrt(deg[N:2 * N, 0:1] + 1.0)
    p = p_ref[...]
    acc = acc_ref[...]
    b1 = b1_ref[...]
    x1 = jnp.maximum(dinv1 * (acc[0:N, 0:H] + p[0:N, 0:H]) + b1, 0.0)
    x2 = jnp.maximum(dinv2 * (acc[N:2 * N, H:2 * H] + p[0:N, H:2 * H]) + b1, 0.0)
    hcat = jnp.concatenate([x1, x2], axis=1)
    g = jnp.dot(hcat, w2_ref[...], preferred_element_type=jnp.float32)
    q_ref[0:N, :] = g * dinv1
    q_ref[N:2 * N, :] = g * dinv2


def _fin_body(acc_ref, q_ref, deg_ref, b2_ref, h2_ref):
    deg = deg_ref[...]
    dinv1 = lax.rsqrt(deg[0:N, 0:1] + 1.0)
    dinv2 = lax.rsqrt(deg[N:2 * N, 0:1] + 1.0)
    q = q_ref[...]
    acc = acc_ref[...]
    b2 = b2_ref[...]
    y1 = jnp.maximum(dinv1 * (acc[0:N] + q[0:N]) + b2, 0.0)
    y2 = jnp.maximum(dinv2 * (acc[N:2 * N] + q[N:2 * N]) + b2, 0.0)
    h2_ref[0:N, 0:2 * H] = y1
    h2_ref[0:N, 2 * H:4 * H] = y2


def _pool_body(h2_ref, batch_ref, wfc_ref, bfc_ref, o_ref):
    h2 = h2_ref[...]
    b = batch_ref[...]
    oh = (b == lax.broadcasted_iota(jnp.int32, (N, G), 1)).astype(jnp.float32)
    sums = lax.dot_general(oh, h2, (((0,), (0,)), ((), ())),
                           preferred_element_type=jnp.float32)
    counts = jnp.sum(oh, axis=0)
    pooled = sums / jnp.maximum(counts, 1.0)[:, None]
    o_ref[...] = jnp.dot(pooled, wfc_ref[...],
                         preferred_element_type=jnp.float32) + bfc_ref[...]


def kernel(x, edge_index1, edge_index2, batch, W1, b1, W2, b2, Wfc, bfc):
    ei1 = edge_index1.astype(jnp.int32)
    ei2 = edge_index2.astype(jnp.int32)
    pad_i = jnp.arange(PADE, dtype=jnp.int32)
    pad_row = pad_i % N
    pad_col = N + pad_i % (ACC_ROWS - N)
    row1_flat = jnp.concatenate([ei1[0], pad_row, ei2[0], pad_row])
    row2_flat = jnp.concatenate([ei1[0], pad_row, ei2[0] + N, pad_row + N])
    col_flat = jnp.concatenate([ei1[1], pad_col, ei2[1], pad_col])
    row1_hbm = row1_flat.reshape(2 * CPS, LANES)
    row2_hbm = row2_flat.reshape(2 * CPS, LANES)
    col_hbm = col_flat.reshape(2 * CPS, LANES)
    ones_h = jnp.ones((LANES, DW), jnp.float32)
    zer_deg = jnp.zeros((ACC_ROWS // NSUB, DW), jnp.float32)
    zer128 = jnp.zeros((ACC_ROWS // NSUB, 2 * H), jnp.float32)

    deg = _get_deg_kernel()(col_hbm, ones_h, zer_deg)
    p = pl.pallas_call(
        _prep_body,
        out_shape=jax.ShapeDtypeStruct((N, 2 * H), jnp.float32),
    )(x, W1, deg)
    acc1 = _make_agg(2 * H)(p, row1_hbm, col_hbm, zer128)
    q = pl.pallas_call(
        _mid_body,
        out_shape=jax.ShapeDtypeStruct((2 * N, 2 * H), jnp.float32),
    )(acc1, p, deg, b1.reshape(1, H), W2)
    acc2 = _make_agg(2 * H)(q, row2_hbm, col_hbm, zer128)
    h2 = pl.pallas_call(
        _fin_body,
        out_shape=jax.ShapeDtypeStruct((N, 4 * H), jnp.float32),
    )(acc2, q, deg, b2.reshape(1, 2 * H))
    out = pl.pallas_call(
        _pool_body,
        out_shape=jax.ShapeDtypeStruct((G, NCLS), jnp.float32),
    )(h2, batch.reshape(N, 1).astype(jnp.int32), Wfc, bfc.reshape(1, NCLS))
    return out


# speedup vs baseline: 27.2244x; 1.1613x over previous
import dataclasses
import functools

import jax
import jax.numpy as jnp
from jax import lax
from jax.experimental import pallas as pl
from jax.experimental.pallas import tpu as pltpu
from jax.experimental.pallas import tpu_sc as plsc

N = 10000
D = 128
H = 64
G = 64
NCLS = 32
E = 320000
LANES = 128
NSUB = 16
CPT = 160
NSLAB = 5
SCH = CPT // NSLAB
CPS = CPT * NSUB
PADE = CPS * LANES - E
ACC_ROWS = NSUB * 640
RPT = 624
DW = 128

@functools.cache
def _get_deg_kernel():
    mesh = plsc.VectorSubcoreMesh(core_axis_name="c", subcore_axis_name="s")

    @functools.partial(
        pl.kernel,
        out_type=jax.ShapeDtypeStruct((2 * NSUB * ACC_ROWS,), jnp.float32),
        mesh=mesh,
        scratch_types=[
            pltpu.VMEM((CPT, LANES), jnp.int32),
            pltpu.VMEM((ACC_ROWS,), jnp.float32),
        ],
        compiler_params=dataclasses.replace(
            pltpu.CompilerParams(), needs_layout_passes=False),
    )
    def _deg_kernel(col_hbm, zeros_hbm, out_hbm, col_v, hist):
        c = lax.axis_index("c")
        s = lax.axis_index("s")
        pltpu.sync_copy(col_hbm.at[pl.ds(c * CPS + s * CPT, CPT)], col_v)
        pltpu.sync_copy(zeros_hbm, hist)
        ones = jnp.ones((16,), jnp.float32)

        @pl.loop(0, CPT)
        def _(j):
            for k in range(LANES // 16):
                idx = col_v[j, pl.ds(k * 16, 16)]
                plsc.addupdate_scatter(hist, [idx], ones)

        wid = c * NSUB + s
        pltpu.sync_copy(hist, out_hbm.at[pl.ds(wid * ACC_ROWS, ACC_ROWS)])

    return _deg_kernel


@functools.cache
def _make_agg(W):
    mesh = plsc.VectorSubcoreMesh(core_axis_name="c", subcore_axis_name="s")

    @functools.partial(
        pl.kernel,
        out_type=jax.ShapeDtypeStruct((2 * N, W), jnp.float32),
        mesh=mesh,
        scratch_types=[
            pltpu.VMEM((SCH, LANES), jnp.int32),
            pltpu.VMEM((SCH, LANES), jnp.int32),
            pltpu.VMEM((LANES, W), jnp.float32),
            pltpu.VMEM((LANES, W), jnp.float32),
            pltpu.VMEM_SHARED((ACC_ROWS, W), jnp.float32),
            pltpu.SemaphoreType.DMA,
        ],
    )
    def _agg(p_hbm, row_hbm, col_hbm, zeros_hbm, out_hbm,
             row_v, col_v, rows_a, rows_b, acc_sh, sem):
        c = lax.axis_index("c")
        s = lax.axis_index("s")
        base = c * CPS + s * CPT
        pltpu.sync_copy(zeros_hbm, acc_sh.at[pl.ds(s * (ACC_ROWS // NSUB), ACC_ROWS // NSUB)])
        plsc.subcore_barrier()

        @pl.loop(0, NSLAB)
        def _(t):
            pltpu.sync_copy(row_hbm.at[pl.ds(base + t * SCH, SCH)], row_v)
            pltpu.sync_copy(col_hbm.at[pl.ds(base + t * SCH, SCH)], col_v)
            pltpu.async_copy(p_hbm.at[row_v.at[0]], rows_a, sem).wait()

            @pl.loop(0, SCH // 2)
            def _(u):
                j0 = 2 * u
                hb = pltpu.async_copy(p_hbm.at[row_v.at[j0 + 1]], rows_b, sem)
                pltpu.sync_copy(rows_a, acc_sh.at[col_v.at[j0]], add=True)
                hb.wait()

                @pl.when(u < SCH // 2 - 1)
                def _():
                    pltpu.async_copy(p_hbm.at[row_v.at[j0 + 2]], rows_a, sem)

                pltpu.sync_copy(rows_b, acc_sh.at[col_v.at[j0 + 1]], add=True)

                @pl.when(u < SCH // 2 - 1)
                def _():
                    pltpu.make_async_copy(
                        p_hbm.at[row_v.at[j0 + 2]], rows_a, sem).wait()

        plsc.subcore_barrier()
        pltpu.sync_copy(acc_sh.at[pl.ds(s * RPT, RPT)],
                        out_hbm.at[pl.ds(c * N + s * RPT, RPT)])

        @pl.when(s == 0)
        def _():
            pltpu.sync_copy(acc_sh.at[pl.ds(NSUB * RPT, N - NSUB * RPT)],
                            out_hbm.at[pl.ds(c * N + NSUB * RPT, N - NSUB * RPT)])

    return _agg


def _degsum_body(degp_ref, o_ref):
    dp = degp_ref[...]
    d1 = jnp.sum(dp[0:NSUB, :], axis=0, keepdims=True) + 1.0
    d2 = jnp.sum(dp[NSUB:2 * NSUB, :], axis=0, keepdims=True) + 1.0
    o_ref[0:1, 0:ACC_ROWS] = lax.rsqrt(d1)
    o_ref[0:1, ACC_ROWS:2 * ACC_ROWS] = lax.rsqrt(d2)


def _prep_body(x_ref, w1_ref, dinv_ref, p_ref):
    h = jnp.dot(x_ref[...], w1_ref[...], preferred_element_type=jnp.float32)
    dc = dinv_ref[...]
    dinv1 = dc[0:N]
    dinv2 = dc[ACC_ROWS:ACC_ROWS + N]
    p_ref[0:N, 0:H] = h * dinv1
    p_ref[0:N, H:2 * H] = h * dinv2


def _mid_body(acc_ref, p_ref, dinv_ref, b1_ref, w2_ref, q_ref):
    dc = dinv_ref[...]
    dinv1 = dc[0:N]
    dinv2 = dc[ACC_ROWS:ACC_ROWS + N]
    p = p_ref[...]
    acc = acc_ref[...]
    b1 = b1_ref[...]
    x1 = jnp.maximum(dinv1 * (acc[0:N, 0:H] + p[0:N, 0:H]) + b1, 0.0)
    x2 = jnp.maximum(dinv2 * (acc[N:2 * N, H:2 * H] + p[0:N, H:2 * H]) + b1, 0.0)
    hcat = jnp.concatenate([x1, x2], axis=1)
    g = jnp.dot(hcat, w2_ref[...], preferred_element_type=jnp.float32)
    q_ref[0:N, :] = g * dinv1
    q_ref[N:2 * N, :] = g * dinv2


def _fin_body(acc_ref, q_ref, dinv_ref, b2_ref, h2_ref):
    dc = dinv_ref[...]
    dinv1 = dc[0:N]
    dinv2 = dc[ACC_ROWS:ACC_ROWS + N]
    q = q_ref[...]
    acc = acc_ref[...]
    b2 = b2_ref[...]
    y1 = jnp.maximum(dinv1 * (acc[0:N] + q[0:N]) + b2, 0.0)
    y2 = jnp.maximum(dinv2 * (acc[N:2 * N] + q[N:2 * N]) + b2, 0.0)
    h2_ref[0:N, 0:2 * H] = y1
    h2_ref[0:N, 2 * H:4 * H] = y2


def _pool_body(h2_ref, batch_ref, wfc_ref, bfc_ref, o_ref):
    h2 = h2_ref[...]
    b = batch_ref[...]
    oh = (b == lax.broadcasted_iota(jnp.int32, (N, G), 1)).astype(jnp.float32)
    sums = lax.dot_general(oh, h2, (((0,), (0,)), ((), ())),
                           preferred_element_type=jnp.float32)
    counts = jnp.sum(oh, axis=0)
    pooled = sums / jnp.maximum(counts, 1.0)[:, None]
    o_ref[...] = jnp.dot(pooled, wfc_ref[...],
                         preferred_element_type=jnp.float32) + bfc_ref[...]


def kernel(x, edge_index1, edge_index2, batch, W1, b1, W2, b2, Wfc, bfc):
    ei1 = edge_index1.astype(jnp.int32)
    ei2 = edge_index2.astype(jnp.int32)
    pad_i = jnp.arange(PADE, dtype=jnp.int32)
    pad_row = pad_i % N
    pad_col = N + pad_i % (ACC_ROWS - N)
    row1_flat = jnp.concatenate([ei1[0], pad_row, ei2[0], pad_row])
    row2_flat = jnp.concatenate([ei1[0], pad_row, ei2[0] + N, pad_row + N])
    col_flat = jnp.concatenate([ei1[1], pad_col, ei2[1], pad_col])
    row1_hbm = row1_flat.reshape(2 * CPS, LANES)
    row2_hbm = row2_flat.reshape(2 * CPS, LANES)
    col_hbm = col_flat.reshape(2 * CPS, LANES)
    zer_hist = jnp.zeros((ACC_ROWS,), jnp.float32)
    zer128 = jnp.zeros((ACC_ROWS // NSUB, 2 * H), jnp.float32)

    degh = _get_deg_kernel()(col_hbm, zer_hist)
    dinv_row = pl.pallas_call(
        _degsum_body,
        out_shape=jax.ShapeDtypeStruct((1, 2 * ACC_ROWS), jnp.float32),
    )(degh.reshape(2 * NSUB, ACC_ROWS))
    dinv = dinv_row.reshape(2 * ACC_ROWS, 1)
    p = pl.pallas_call(
        _prep_body,
        out_shape=jax.ShapeDtypeStruct((N, 2 * H), jnp.float32),
    )(x, W1, dinv)
    acc1 = _make_agg(2 * H)(p, row1_hbm, col_hbm, zer128)
    q = pl.pallas_call(
        _mid_body,
        out_shape=jax.ShapeDtypeStruct((2 * N, 2 * H), jnp.float32),
    )(acc1, p, dinv, b1.reshape(1, H), W2)
    acc2 = _make_agg(2 * H)(q, row2_hbm, col_hbm, zer128)
    h2 = pl.pallas_call(
        _fin_body,
        out_shape=jax.ShapeDtypeStruct((N, 4 * H), jnp.float32),
    )(acc2, q, dinv, b2.reshape(1, 2 * H))
    out = pl.pallas_call(
        _pool_body,
        out_shape=jax.ShapeDtypeStruct((G, NCLS), jnp.float32),
    )(h2, batch.reshape(N, 1).astype(jnp.int32), Wfc, bfc.reshape(1, NCLS))
    return out


# speedup vs baseline: 27.6108x; 1.0142x over previous
import dataclasses
import functools

import jax
import jax.numpy as jnp
from jax import lax
from jax.experimental import pallas as pl
from jax.experimental.pallas import tpu as pltpu
from jax.experimental.pallas import tpu_sc as plsc

N = 10000
D = 128
H = 64
G = 64
NCLS = 32
E = 320000
LANES = 128
NSUB = 16
CPT = 160
NSLAB = 5
SCH = CPT // NSLAB
CPS = CPT * NSUB
PADE = CPS * LANES - E
ACC_ROWS = NSUB * 640
RPT = 624
DW = 128

@functools.cache
def _get_deg_kernel():
    mesh = plsc.VectorSubcoreMesh(core_axis_name="c", subcore_axis_name="s")

    @functools.partial(
        pl.kernel,
        out_type=jax.ShapeDtypeStruct((2 * NSUB * ACC_ROWS,), jnp.float32),
        mesh=mesh,
        scratch_types=[
            pltpu.VMEM((CPT, LANES), jnp.int32),
            pltpu.VMEM((ACC_ROWS,), jnp.float32),
        ],
        compiler_params=dataclasses.replace(
            pltpu.CompilerParams(), needs_layout_passes=False),
    )
    def _deg_kernel(col_hbm, zeros_hbm, out_hbm, col_v, hist):
        c = lax.axis_index("c")
        s = lax.axis_index("s")
        pltpu.sync_copy(col_hbm.at[pl.ds(c * CPS + s * CPT, CPT)], col_v)
        pltpu.sync_copy(zeros_hbm, hist)
        ones = jnp.ones((16,), jnp.float32)

        @pl.loop(0, CPT)
        def _(j):
            for k in range(LANES // 16):
                idx = col_v[j, pl.ds(k * 16, 16)]
                plsc.addupdate_scatter(hist, [idx], ones)

        wid = c * NSUB + s
        pltpu.sync_copy(hist, out_hbm.at[pl.ds(wid * ACC_ROWS, ACC_ROWS)])

    return _deg_kernel


@functools.cache
def _make_agg(W):
    mesh = plsc.VectorSubcoreMesh(core_axis_name="c", subcore_axis_name="s")

    @functools.partial(
        pl.kernel,
        out_type=jax.ShapeDtypeStruct((2 * N, W), jnp.float32),
        mesh=mesh,
        scratch_types=[
            pltpu.VMEM((SCH, LANES), jnp.int32),
            pltpu.VMEM((SCH, LANES), jnp.int32),
            pltpu.VMEM((LANES, W), jnp.float32),
            pltpu.VMEM((LANES, W), jnp.float32),
            pltpu.VMEM_SHARED((ACC_ROWS, W), jnp.float32),
            pltpu.SemaphoreType.DMA,
        ],
    )
    def _agg(p_hbm, row_hbm, col_hbm, zeros_hbm, out_hbm,
             row_v, col_v, rows_a, rows_b, acc_sh, sem):
        c = lax.axis_index("c")
        s = lax.axis_index("s")
        base = c * CPS + s * CPT
        pltpu.sync_copy(zeros_hbm, acc_sh.at[pl.ds(s * (ACC_ROWS // NSUB), ACC_ROWS // NSUB)])
        plsc.subcore_barrier()

        @pl.loop(0, NSLAB)
        def _(t):
            pltpu.sync_copy(row_hbm.at[pl.ds(base + t * SCH, SCH)], row_v)
            pltpu.sync_copy(col_hbm.at[pl.ds(base + t * SCH, SCH)], col_v)
            pltpu.async_copy(p_hbm.at[row_v.at[0]], rows_a, sem).wait()

            @pl.loop(0, SCH // 2)
            def _(u):
                j0 = 2 * u
                hb = pltpu.async_copy(p_hbm.at[row_v.at[j0 + 1]], rows_b, sem)
                pltpu.sync_copy(rows_a, acc_sh.at[col_v.at[j0]], add=True)
                hb.wait()

                @pl.when(u < SCH // 2 - 1)
                def _():
                    pltpu.async_copy(p_hbm.at[row_v.at[j0 + 2]], rows_a, sem)

                pltpu.sync_copy(rows_b, acc_sh.at[col_v.at[j0 + 1]], add=True)

                @pl.when(u < SCH // 2 - 1)
                def _():
                    pltpu.make_async_copy(
                        p_hbm.at[row_v.at[j0 + 2]], rows_a, sem).wait()

        plsc.subcore_barrier()
        pltpu.sync_copy(acc_sh.at[pl.ds(s * RPT, RPT)],
                        out_hbm.at[pl.ds(c * N + s * RPT, RPT)])

        @pl.when(s == 0)
        def _():
            pltpu.sync_copy(acc_sh.at[pl.ds(NSUB * RPT, N - NSUB * RPT)],
                            out_hbm.at[pl.ds(c * N + NSUB * RPT, N - NSUB * RPT)])

    return _agg


def _degsum_body(degp_ref, o_ref):
    dp = degp_ref[...]
    d1 = jnp.sum(dp[0:NSUB, :], axis=0, keepdims=True) + 1.0
    d2 = jnp.sum(dp[NSUB:2 * NSUB, :], axis=0, keepdims=True) + 1.0
    o_ref[0:1, 0:ACC_ROWS] = lax.rsqrt(d1)
    o_ref[0:1, ACC_ROWS:2 * ACC_ROWS] = lax.rsqrt(d2)


def _matmul1_body(x_ref, w1_ref, h_ref):
    h_ref[...] = jnp.dot(x_ref[...], w1_ref[...],
                         preferred_element_type=jnp.float32)


def _prep_body(h_ref, dinv_ref, p_ref):
    h = h_ref[...]
    dc = dinv_ref[...]
    dinv1 = dc[0:N]
    dinv2 = dc[ACC_ROWS:ACC_ROWS + N]
    p_ref[0:N, 0:H] = h * dinv1
    p_ref[0:N, H:2 * H] = h * dinv2


def _mid_body(acc_ref, p_ref, dinv_ref, b1_ref, w2_ref, q_ref):
    dc = dinv_ref[...]
    dinv1 = dc[0:N]
    dinv2 = dc[ACC_ROWS:ACC_ROWS + N]
    p = p_ref[...]
    acc = acc_ref[...]
    b1 = b1_ref[...]
    x1 = jnp.maximum(dinv1 * (acc[0:N, 0:H] + p[0:N, 0:H]) + b1, 0.0)
    x2 = jnp.maximum(dinv2 * (acc[N:2 * N, H:2 * H] + p[0:N, H:2 * H]) + b1, 0.0)
    hcat = jnp.concatenate([x1, x2], axis=1)
    g = jnp.dot(hcat, w2_ref[...], preferred_element_type=jnp.float32)
    q_ref[0:N, :] = g * dinv1
    q_ref[N:2 * N, :] = g * dinv2


def _fin_body(acc_ref, q_ref, dinv_ref, b2_ref, batch_ref, wfc_ref, bfc_ref,
              o_ref):
    dc = dinv_ref[...]
    dinv1 = dc[0:N]
    dinv2 = dc[ACC_ROWS:ACC_ROWS + N]
    q = q_ref[...]
    acc = acc_ref[...]
    b2 = b2_ref[...]
    y1 = jnp.maximum(dinv1 * (acc[0:N] + q[0:N]) + b2, 0.0)
    y2 = jnp.maximum(dinv2 * (acc[N:2 * N] + q[N:2 * N]) + b2, 0.0)
    b = batch_ref[...]
    oh = (b == lax.broadcasted_iota(jnp.int32, (N, G), 1)).astype(jnp.float32)
    sums1 = lax.dot_general(oh, y1, (((0,), (0,)), ((), ())),
                            preferred_element_type=jnp.float32)
    sums2 = lax.dot_general(oh, y2, (((0,), (0,)), ((), ())),
                            preferred_element_type=jnp.float32)
    counts = jnp.sum(oh, axis=0)
    inv = 1.0 / jnp.maximum(counts, 1.0)[:, None]
    pooled = jnp.concatenate([sums1 * inv, sums2 * inv], axis=1)
    o_ref[...] = jnp.dot(pooled, wfc_ref[...],
                         preferred_element_type=jnp.float32) + bfc_ref[...]


def kernel(x, edge_index1, edge_index2, batch, W1, b1, W2, b2, Wfc, bfc):
    ei1 = edge_index1.astype(jnp.int32)
    ei2 = edge_index2.astype(jnp.int32)
    pad_i = jnp.arange(PADE, dtype=jnp.int32)
    pad_row = pad_i % N
    pad_col = N + pad_i % (ACC_ROWS - N)
    row1_flat = jnp.concatenate([ei1[0], pad_row, ei2[0], pad_row])
    row2_flat = jnp.concatenate([ei1[0], pad_row, ei2[0] + N, pad_row + N])
    col_flat = jnp.concatenate([ei1[1], pad_col, ei2[1], pad_col])
    row1_hbm = row1_flat.reshape(2 * CPS, LANES)
    row2_hbm = row2_flat.reshape(2 * CPS, LANES)
    col_hbm = col_flat.reshape(2 * CPS, LANES)
    zer_hist = jnp.zeros((ACC_ROWS,), jnp.float32)
    zer128 = jnp.zeros((ACC_ROWS // NSUB, 2 * H), jnp.float32)

    degh = _get_deg_kernel()(col_hbm, zer_hist)
    h = pl.pallas_call(
        _matmul1_body,
        out_shape=jax.ShapeDtypeStruct((N, H), jnp.float32),
    )(x, W1)
    dinv_row = pl.pallas_call(
        _degsum_body,
        out_shape=jax.ShapeDtypeStruct((1, 2 * ACC_ROWS), jnp.float32),
    )(degh.reshape(2 * NSUB, ACC_ROWS))
    dinv = dinv_row.reshape(2 * ACC_ROWS, 1)
    p = pl.pallas_call(
        _prep_body,
        out_shape=jax.ShapeDtypeStruct((N, 2 * H), jnp.float32),
    )(h, dinv)
    acc1 = _make_agg(2 * H)(p, row1_hbm, col_hbm, zer128)
    q = pl.pallas_call(
        _mid_body,
        out_shape=jax.ShapeDtypeStruct((2 * N, 2 * H), jnp.float32),
    )(acc1, p, dinv, b1.reshape(1, H), W2)
    acc2 = _make_agg(2 * H)(q, row2_hbm, col_hbm, zer128)
    out = pl.pallas_call(
        _fin_body,
        out_shape=jax.ShapeDtypeStruct((G, NCLS), jnp.float32),
    )(acc2, q, dinv, b2.reshape(1, 2 * H),
      batch.reshape(N, 1).astype(jnp.int32), Wfc, bfc.reshape(1, NCLS))
    return out
